# baseline (speedup 1.0000x reference)
import dataclasses
import functools

import jax
import jax.numpy as jnp
from jax import lax
from jax.experimental import pallas as pl
from jax.experimental.pallas import tpu as pltpu
from jax.experimental.pallas import tpu_sc as plsc

_BATCH = 16384
_DIM = 32
_LANES = 16
_NUM_CORES = 2
_NUM_SUBCORES = 16
_NUM_WORKERS = _NUM_CORES * _NUM_SUBCORES
_PER_WORKER = _BATCH // _NUM_WORKERS
_V = 1000000
_TAILB = 999936
_TAILN = _V - _TAILB
_SHARD = 31360
_W = 512
_CLAMP = _TAILB - _W
_WLCAP = 1152
_JROWS = _BATCH + 128
_JSIZE = _JROWS * _DIM
_NSCAT = _WLCAP * _DIM // 128


def _mesh_and_params():
    mesh = plsc.VectorSubcoreMesh(core_axis_name="c", subcore_axis_name="s")
    cp = pltpu.CompilerParams()
    if "needs_layout_passes" in pltpu.CompilerParams.__dataclass_fields__:
        cp = dataclasses.replace(cp, needs_layout_passes=False)
    return mesh, cp


def _make_k1():
    mesh, cp = _mesh_and_params()

    @functools.partial(
        pl.kernel,
        mesh=mesh,
        compiler_params=cp,
        out_type=[jax.ShapeDtypeStruct((_JSIZE,), jnp.float32),
                  jax.ShapeDtypeStruct((_JSIZE,), jnp.float32)],
        scratch_types=[
            pltpu.VMEM((_BATCH,), jnp.int32),
            pltpu.VMEM((_WLCAP,), jnp.int32),
            pltpu.VMEM((16, 128), jnp.int32),
            pltpu.VMEM((2, _DIM, _W), jnp.float32),
            pltpu.VMEM((_WLCAP * _DIM,), jnp.float32),
            pltpu.VMEM((8, 1, 128), jnp.int32),
            pltpu.VMEM((2048,), jnp.float32),
            pltpu.SMEM((1,), jnp.int32),
            pltpu.SemaphoreType.DMA,
            pltpu.SemaphoreType.DMA,
        ],
    )
    def k1(uid_hbm, mid_hbm, ut_hbm, mt_hbm, utail_hbm, mtail_hbm,
           uj_hbm, mj_hbm, idsv, wlc, wlp, slab, rowbuf, idxr, tailv,
           cnt, sem, sem2):
        wid = lax.axis_index("s") * _NUM_CORES + lax.axis_index("c")
        lo = wid * _SHARD
        hi = lax.min(lo + _SHARD, _V)
        hi_ch = lax.min(lo + _SHARD, _TAILB)
        nch = (hi_ch - lo + _W - 1) // _W
        lanes = lax.iota(jnp.int32, _LANES)

        for ids_hbm, embt_hbm, tl_hbm, out_hbm in (
                (uid_hbm, ut_hbm, utail_hbm, uj_hbm),
                (mid_hbm, mt_hbm, mtail_hbm, mj_hbm)):
            pltpu.sync_copy(ids_hbm, idsv)
            pltpu.sync_copy(tl_hbm, tailv)

            @pl.loop(0, _WLCAP // _LANES)
            def _(i):
                wlc[pl.ds(i * _LANES, _LANES)] = jnp.full(
                    (_LANES,), -1, jnp.int32)

            @pl.loop(0, 16)
            def _(r):
                for q in range(8):
                    wlp[r, pl.ds(q * _LANES, _LANES)] = (
                        _BATCH + ((r * 8 + q) % 8) * _LANES + lanes)

            cnt[0] = 0

            @pl.loop(0, _BATCH // _LANES)
            def _(g):
                idv = idsv[pl.ds(g * _LANES, _LANES)]
                msk = (idv >= lo) & (idv < hi)
                mi = jnp.where(msk, 1, 0).astype(jnp.int32)
                cs = plsc.cumsum(mi)
                base = cnt[0]
                pos16 = base + cs - mi
                plsc.store_scatter(wlc, [pos16], idv, mask=msk)
                plsc.store_scatter(wlp, [pos16 >> 7, pos16 & 127],
                                   g * _LANES + lanes, mask=msk)
                cnt[0] = base + jnp.sum(mi)

            n = cnt[0]
            ngrp = (n + _LANES - 1) // _LANES

            def c0_of(k):
                return lax.min(lo + k * _W, _CLAMP)

            def fire(k):
                pltpu.async_copy(
                    embt_hbm.at[:, pl.ds(c0_of(k), _W)],
                    slab.at[k & 1], sem)

            def drain(k):
                pltpu.make_async_copy(
                    embt_hbm.at[:, pl.ds(c0_of(k), _W)],
                    slab.at[k & 1], sem).wait()

            def extract(localv, sel, slotv, gathered):
                for d in range(_DIM):
                    vd = gathered(d, localv, sel)
                    plsc.store_scatter(
                        rowbuf, [slotv * _DIM + d], vd, mask=sel)

            fire(0)

            @pl.loop(0, nch)
            def _(k):
                @pl.when(k + 1 < nch)
                def _():
                    fire(k + 1)
                drain(k)
                c0n = lo + k * _W
                c0 = c0_of(k)
                buf16 = jnp.full((_LANES,), k & 1, jnp.int32)

                @pl.loop(0, ngrp)
                def _(gi):
                    colv = wlc[pl.ds(gi * _LANES, _LANES)]
                    sel = ((colv >= c0n) & (colv < c0n + _W)
                           & (colv < _TAILB))
                    npop = jnp.sum(jnp.where(sel, 1, 0).astype(jnp.int32))

                    @pl.when(npop > 0)
                    def _():
                        slotv = gi * _LANES + lanes
                        localv = jnp.where(sel, colv - c0, 0)
                        extract(localv, sel, slotv,
                                lambda d, lv, s: plsc.load_gather(
                                    slab,
                                    [buf16, jnp.full((_LANES,), d,
                                                     jnp.int32), lv],
                                    mask=s))

            @pl.loop(0, ngrp)
            def _(gi):
                colv = wlc[pl.ds(gi * _LANES, _LANES)]
                sel = colv >= _TAILB
                npop = jnp.sum(jnp.where(sel, 1, 0).astype(jnp.int32))

                @pl.when(npop > 0)
                def _():
                    slotv = gi * _LANES + lanes
                    localv = jnp.where(sel, colv - _TAILB, 0)
                    extract(localv, sel, slotv,
                            lambda d, lv, s: plsc.load_gather(
                                tailv, [d * _TAILN + lv], mask=s))

            def scat_transfer(jj, ring):
                return pltpu.make_async_copy(
                    rowbuf.at[pl.ds(jj * 128, 128)],
                    out_hbm.at[idxr.at[ring, 0]], sem2)

            @pl.loop(0, _NSCAT)
            def _(jj):
                ring = jj & 7

                @pl.when(jj >= 8)
                def _():
                    scat_transfer(jj - 8, ring).wait()

                for q in range(8):
                    flat = jj * 128 + q * _LANES + lanes
                    slot = flat >> 5
                    off = flat & (_DIM - 1)
                    pv = plsc.load_gather(wlp, [slot >> 7, slot & 127])
                    idxr[ring, 0, pl.ds(q * _LANES, _LANES)] = (
                        pv * _DIM + off)
                scat_transfer(jj, ring).start()

            @pl.loop(0, 8)
            def _(r):
                scat_transfer(_NSCAT - 8 + r, (_NSCAT - 8 + r) & 7).wait()

    return k1


def _make_k2():
    mesh, cp = _mesh_and_params()

    @functools.partial(
        pl.kernel,
        mesh=mesh,
        compiler_params=cp,
        out_type=jax.ShapeDtypeStruct((_BATCH,), jnp.float32),
        scratch_types=[
            pltpu.VMEM((4, 128), jnp.int32),
            pltpu.VMEM((4, 128), jnp.int32),
            pltpu.VMEM((_PER_WORKER * _DIM,), jnp.float32),
            pltpu.VMEM((_PER_WORKER * _DIM,), jnp.float32),
            pltpu.VMEM((_PER_WORKER,), jnp.float32),
            pltpu.VMEM((_PER_WORKER,), jnp.float32),
            pltpu.VMEM((_PER_WORKER,), jnp.float32),
            pltpu.SemaphoreType.DMA,
            pltpu.SemaphoreType.DMA,
        ],
    )
    def k2(uid_hbm, mid_hbm, uj_hbm, mj_hbm, ubias_hbm, mbias_hbm,
           out_hbm, uidx, midx, uv, mv, ubv, mbv, outv, sem, sem2):
        wid = lax.axis_index("s") * _NUM_CORES + lax.axis_index("c")
        base = wid * _PER_WORKER
        lanes = lax.iota(jnp.int32, _LANES)

        pltpu.sync_copy(uid_hbm.at[wid], uidx)
        pltpu.sync_copy(mid_hbm.at[wid], midx)

        bias_copies = []
        for j in range(4):
            sl = pl.ds(j * 128, 128)
            bias_copies.append(pltpu.async_copy(
                ubias_hbm.at[uidx.at[j]], ubv.at[sl], sem2))
            bias_copies.append(pltpu.async_copy(
                mbias_hbm.at[midx.at[j]], mbv.at[sl], sem2))

        pltpu.sync_copy(uj_hbm.at[pl.ds(base * _DIM, _PER_WORKER * _DIM)],
                        uv)
        pltpu.sync_copy(mj_hbm.at[pl.ds(base * _DIM, _PER_WORKER * _DIM)],
                        mv)

        @pl.loop(0, _PER_WORKER // _LANES)
        def _(g):
            acc = jnp.zeros((_LANES,), jnp.float32)
            for j in range(_LANES):
                r = (g * _LANES + j) * _DIM
                s = (uv[pl.ds(r, _LANES)] * mv[pl.ds(r, _LANES)]
                     + uv[pl.ds(r + _LANES, _LANES)]
                     * mv[pl.ds(r + _LANES, _LANES)])
                acc = jnp.where(lanes == j, jnp.sum(s), acc)
            outv[pl.ds(g * _LANES, _LANES)] = acc

        for bc in bias_copies:
            bc.wait()
        for h in range(_PER_WORKER // _LANES):
            sl = pl.ds(h * _LANES, _LANES)
            outv[sl] = outv[sl] + ubv[sl] + mbv[sl]

        pltpu.sync_copy(outv, out_hbm.at[pl.ds(base, _PER_WORKER)])

    return k2


def kernel(user_ids, movie_ids, user_emb, movie_emb, user_bias, movie_bias):
    uids = user_ids.astype(jnp.int32)
    mids = movie_ids.astype(jnp.int32)
    ut = user_emb.T
    mt = movie_emb.T
    utail = ut[:, _TAILB:].reshape(-1)
    mtail = mt[:, _TAILB:].reshape(-1)
    uj, mj = _make_k1()(uids, mids, ut, mt, utail, mtail)
    return _make_k2()(
        uids.reshape(_NUM_WORKERS, 4, 128),
        mids.reshape(_NUM_WORKERS, 4, 128),
        uj, mj, user_bias.reshape(-1), movie_bias.reshape(-1))


# speedup vs baseline: 1.1815x; 1.1815x over previous
import dataclasses
import functools

import jax
import jax.numpy as jnp
from jax import lax
from jax.experimental import pallas as pl
from jax.experimental.pallas import tpu as pltpu
from jax.experimental.pallas import tpu_sc as plsc

_BATCH = 16384
_DIM = 32
_LANES = 16
_NUM_CORES = 2
_NUM_SUBCORES = 16
_NUM_WORKERS = _NUM_CORES * _NUM_SUBCORES
_PER_WORKER = _BATCH // _NUM_WORKERS
_V = 1000000
_TAILB = 999936
_TAILN = _V - _TAILB
_SHARD = 31360
_W = 1024
_CLAMP = _TAILB - _W
_WLCAP = 1024
_JROWS = _BATCH + 128
_JSIZE = _JROWS * _DIM
_NSCAT = _WLCAP * _DIM // 128


def _mesh_and_params():
    mesh = plsc.VectorSubcoreMesh(core_axis_name="c", subcore_axis_name="s")
    cp = pltpu.CompilerParams()
    if "needs_layout_passes" in pltpu.CompilerParams.__dataclass_fields__:
        cp = dataclasses.replace(cp, needs_layout_passes=False)
    return mesh, cp


def _make_k1():
    mesh, cp = _mesh_and_params()

    @functools.partial(
        pl.kernel,
        mesh=mesh,
        compiler_params=cp,
        out_type=[jax.ShapeDtypeStruct((_JSIZE,), jnp.float32),
                  jax.ShapeDtypeStruct((_JSIZE,), jnp.float32)],
        scratch_types=[
            pltpu.VMEM((_BATCH,), jnp.int32),
            pltpu.VMEM((_WLCAP,), jnp.int32),
            pltpu.VMEM((8, 128), jnp.int32),
            pltpu.VMEM((2, 4, 8, _W), jnp.float32),
            pltpu.VMEM((_WLCAP * _DIM,), jnp.float32),
            pltpu.VMEM((8, 1, 128), jnp.int32),
            pltpu.VMEM((2048,), jnp.float32),
            pltpu.SMEM((1,), jnp.int32),
            pltpu.SemaphoreType.DMA,
            pltpu.SemaphoreType.DMA,
        ],
    )
    def k1(uid_hbm, mid_hbm, ut_hbm, mt_hbm, utail_hbm, mtail_hbm,
           uj_hbm, mj_hbm, idsv, wlc, wlp, slab, rowbuf, idxr, tailv,
           cnt, sem, sem2):
        wid = lax.axis_index("s") * _NUM_CORES + lax.axis_index("c")
        lo = wid * _SHARD
        hi = lax.min(lo + _SHARD, _V)
        hi_ch = lax.min(lo + _SHARD, _TAILB)
        nch = (hi_ch - lo + _W - 1) // _W
        lanes = lax.iota(jnp.int32, _LANES)

        for ids_hbm, embt_hbm, tl_hbm, out_hbm in (
                (uid_hbm, ut_hbm, utail_hbm, uj_hbm),
                (mid_hbm, mt_hbm, mtail_hbm, mj_hbm)):
            pltpu.sync_copy(ids_hbm, idsv)
            pltpu.sync_copy(tl_hbm, tailv)

            @pl.loop(0, _WLCAP // _LANES)
            def _(i):
                wlc[pl.ds(i * _LANES, _LANES)] = jnp.full(
                    (_LANES,), -1, jnp.int32)

            @pl.loop(0, 8)
            def _(r):
                for q in range(8):
                    wlp[r, pl.ds(q * _LANES, _LANES)] = (
                        _BATCH + ((r * 8 + q) % 8) * _LANES + lanes)

            cnt[0] = 0

            @pl.loop(0, _BATCH // _LANES)
            def _(g):
                idv = idsv[pl.ds(g * _LANES, _LANES)]
                msk = (idv >= lo) & (idv < hi)
                mi = jnp.where(msk, 1, 0).astype(jnp.int32)
                cs = plsc.cumsum(mi)
                base = cnt[0]
                pos16 = base + cs - mi
                plsc.store_scatter(wlc, [pos16], idv, mask=msk)
                plsc.store_scatter(wlp, [pos16 >> 7, pos16 & 127],
                                   g * _LANES + lanes, mask=msk)
                cnt[0] = base + cs[_LANES - 1]

            n = cnt[0]
            ngrp = (n + _LANES - 1) // _LANES

            def c0_of(k):
                return lax.min(lo + k * _W, _CLAMP)

            def fire(k):
                for i in range(4):
                    pltpu.async_copy(
                        embt_hbm.at[i, :, pl.ds(c0_of(k), _W)],
                        slab.at[k & 1, i], sem)

            def drain(k):
                for i in range(4):
                    pltpu.make_async_copy(
                        embt_hbm.at[i, :, pl.ds(c0_of(k), _W)],
                        slab.at[k & 1, i], sem).wait()

            def extract(localv, sel, slotv, gathered):
                for d in range(_DIM):
                    vd = gathered(d, localv, sel)
                    plsc.store_scatter(
                        rowbuf, [slotv * _DIM + d], vd, mask=sel)

            fire(0)

            @pl.loop(0, nch)
            def _(k):
                @pl.when(k + 1 < nch)
                def _():
                    fire(k + 1)
                drain(k)
                c0n = lo + k * _W
                c0 = c0_of(k)
                buf16 = jnp.full((_LANES,), k & 1, jnp.int32)

                @pl.loop(0, ngrp)
                def _(gi):
                    colv = wlc[pl.ds(gi * _LANES, _LANES)]
                    sel = ((colv >= c0n) & (colv < c0n + _W)
                           & (colv < _TAILB))
                    npop = plsc.all_reduce_population_count(sel)[0]

                    @pl.when(npop > 0)
                    def _():
                        slotv = gi * _LANES + lanes
                        localv = jnp.where(sel, colv - c0, 0)
                        extract(localv, sel, slotv,
                                lambda d, lv, s: plsc.load_gather(
                                    slab,
                                    [buf16,
                                     jnp.full((_LANES,), d >> 3, jnp.int32),
                                     jnp.full((_LANES,), d & 7, jnp.int32),
                                     lv],
                                    mask=s))

            @pl.loop(0, ngrp)
            def _(gi):
                colv = wlc[pl.ds(gi * _LANES, _LANES)]
                sel = colv >= _TAILB
                npop = plsc.all_reduce_population_count(sel)[0]

                @pl.when(npop > 0)
                def _():
                    slotv = gi * _LANES + lanes
                    localv = jnp.where(sel, colv - _TAILB, 0)
                    extract(localv, sel, slotv,
                            lambda d, lv, s: plsc.load_gather(
                                tailv, [d * _TAILN + lv], mask=s))

            def scat_transfer(jj, ring):
                return pltpu.make_async_copy(
                    rowbuf.at[pl.ds(jj * 128, 128)],
                    out_hbm.at[idxr.at[ring, 0]], sem2)

            @pl.loop(0, _NSCAT)
            def _(jj):
                ring = jj & 7

                @pl.when(jj >= 8)
                def _():
                    scat_transfer(jj - 8, ring).wait()

                for q in range(8):
                    flat = jj * 128 + q * _LANES + lanes
                    slot = flat >> 5
                    off = flat & (_DIM - 1)
                    pv = plsc.load_gather(wlp, [slot >> 7, slot & 127])
                    idxr[ring, 0, pl.ds(q * _LANES, _LANES)] = (
                        pv * _DIM + off)
                scat_transfer(jj, ring).start()

            @pl.loop(0, 8)
            def _(r):
                scat_transfer(_NSCAT - 8 + r, (_NSCAT - 8 + r) & 7).wait()

    return k1


def _make_k2():
    mesh, cp = _mesh_and_params()

    @functools.partial(
        pl.kernel,
        mesh=mesh,
        compiler_params=cp,
        out_type=jax.ShapeDtypeStruct((_BATCH,), jnp.float32),
        scratch_types=[
            pltpu.VMEM((4, 128), jnp.int32),
            pltpu.VMEM((4, 128), jnp.int32),
            pltpu.VMEM((_PER_WORKER * _DIM,), jnp.float32),
            pltpu.VMEM((_PER_WORKER * _DIM,), jnp.float32),
            pltpu.VMEM((_PER_WORKER,), jnp.float32),
            pltpu.VMEM((_PER_WORKER,), jnp.float32),
            pltpu.VMEM((_PER_WORKER,), jnp.float32),
            pltpu.SemaphoreType.DMA,
            pltpu.SemaphoreType.DMA,
        ],
    )
    def k2(uid_hbm, mid_hbm, uj_hbm, mj_hbm, ubias_hbm, mbias_hbm,
           out_hbm, uidx, midx, uv, mv, ubv, mbv, outv, sem, sem2):
        wid = lax.axis_index("s") * _NUM_CORES + lax.axis_index("c")
        base = wid * _PER_WORKER
        lanes = lax.iota(jnp.int32, _LANES)

        pltpu.sync_copy(uid_hbm.at[wid], uidx)
        pltpu.sync_copy(mid_hbm.at[wid], midx)

        bias_copies = []
        for j in range(4):
            sl = pl.ds(j * 128, 128)
            bias_copies.append(pltpu.async_copy(
                ubias_hbm.at[uidx.at[j]], ubv.at[sl], sem2))
            bias_copies.append(pltpu.async_copy(
                mbias_hbm.at[midx.at[j]], mbv.at[sl], sem2))

        pltpu.sync_copy(uj_hbm.at[pl.ds(base * _DIM, _PER_WORKER * _DIM)],
                        uv)
        pltpu.sync_copy(mj_hbm.at[pl.ds(base * _DIM, _PER_WORKER * _DIM)],
                        mv)

        @pl.loop(0, _PER_WORKER // _LANES)
        def _(g):
            acc = jnp.zeros((_LANES,), jnp.float32)
            for j in range(_LANES):
                r = (g * _LANES + j) * _DIM
                s = (uv[pl.ds(r, _LANES)] * mv[pl.ds(r, _LANES)]
                     + uv[pl.ds(r + _LANES, _LANES)]
                     * mv[pl.ds(r + _LANES, _LANES)])
                acc = jnp.where(lanes == j, jnp.sum(s), acc)
            outv[pl.ds(g * _LANES, _LANES)] = acc

        for bc in bias_copies:
            bc.wait()
        for h in range(_PER_WORKER // _LANES):
            sl = pl.ds(h * _LANES, _LANES)
            outv[sl] = outv[sl] + ubv[sl] + mbv[sl]

        pltpu.sync_copy(outv, out_hbm.at[pl.ds(base, _PER_WORKER)])

    return k2


def kernel(user_ids, movie_ids, user_emb, movie_emb, user_bias, movie_bias):
    uids = user_ids.astype(jnp.int32)
    mids = movie_ids.astype(jnp.int32)
    ut = user_emb.T
    mt = movie_emb.T
    utail = ut[:, _TAILB:].reshape(-1)
    mtail = mt[:, _TAILB:].reshape(-1)
    ut3 = ut.reshape(4, 8, _V)
    mt3 = mt.reshape(4, 8, _V)
    uj, mj = _make_k1()(uids, mids, ut3, mt3, utail, mtail)
    return _make_k2()(
        uids.reshape(_NUM_WORKERS, 4, 128),
        mids.reshape(_NUM_WORKERS, 4, 128),
        uj, mj, user_bias.reshape(-1), movie_bias.reshape(-1))


# speedup vs baseline: 2.7781x; 2.3513x over previous
import dataclasses
import functools

import jax
import jax.numpy as jnp
from jax import lax
from jax.experimental import pallas as pl
from jax.experimental.pallas import tpu as pltpu
from jax.experimental.pallas import tpu_sc as plsc

_BATCH = 16384
_DIM = 32
_LANES = 16
_NUM_CORES = 2
_NUM_SUBCORES = 16
_NUM_WORKERS = _NUM_CORES * _NUM_SUBCORES
_PER_WORKER = _BATCH // _NUM_WORKERS
_V = 1000000
_TAILB = 999936
_TAILN = _V - _TAILB
_SHARD = 31360
_W = 1024
_CLAMP = _TAILB - _W
_WLCAP = 1024
_JROWS = _BATCH + 128
_JSIZE = _JROWS * _DIM
_NSCAT = _WLCAP * _DIM // 128


def _mesh_and_params(tc_tiling=True):
    mesh = plsc.VectorSubcoreMesh(core_axis_name="c", subcore_axis_name="s")
    cp = pltpu.CompilerParams()
    if "needs_layout_passes" in pltpu.CompilerParams.__dataclass_fields__:
        cp = dataclasses.replace(cp, needs_layout_passes=False)
    if not tc_tiling and (
            "use_tc_tiling_on_sc" in pltpu.CompilerParams.__dataclass_fields__):
        cp = dataclasses.replace(cp, use_tc_tiling_on_sc=False)
    return mesh, cp


def _make_k1():
    mesh, cp = _mesh_and_params()

    @functools.partial(
        pl.kernel,
        mesh=mesh,
        compiler_params=cp,
        out_type=[
            jax.ShapeDtypeStruct((_NUM_WORKERS * _WLCAP * _DIM,),
                                 jnp.float32),
            jax.ShapeDtypeStruct((_NUM_WORKERS * _WLCAP * _DIM,),
                                 jnp.float32),
            jax.ShapeDtypeStruct((_JROWS,), jnp.int32),
            jax.ShapeDtypeStruct((_JROWS,), jnp.int32),
        ],
        scratch_types=[
            pltpu.VMEM((_BATCH,), jnp.int32),
            pltpu.VMEM((_WLCAP,), jnp.int32),
            pltpu.VMEM((8, 128), jnp.int32),
            pltpu.VMEM((2, 4, 8, _W), jnp.float32),
            pltpu.VMEM((_WLCAP * _DIM,), jnp.float32),
            pltpu.VMEM((_WLCAP,), jnp.int32),
            pltpu.VMEM((2048,), jnp.float32),
            pltpu.SMEM((1,), jnp.int32),
            pltpu.SemaphoreType.DMA,
            pltpu.SemaphoreType.DMA,
        ],
    )
    def k1(uid_hbm, mid_hbm, ut_hbm, mt_hbm, utail_hbm, mtail_hbm,
           uj_hbm, mj_hbm, usm_hbm, msm_hbm, idsv, wlc, wlp, slab, rowbuf,
           valv, tailv, cnt, sem, sem2):
        wid = lax.axis_index("s") * _NUM_CORES + lax.axis_index("c")
        lo = wid * _SHARD
        hi = lax.min(lo + _SHARD, _V)
        hi_ch = lax.min(lo + _SHARD, _TAILB)
        nch = (hi_ch - lo + _W - 1) // _W
        lanes = lax.iota(jnp.int32, _LANES)

        for ids_hbm, embt_hbm, tl_hbm, out_hbm, sm_hbm in (
                (uid_hbm, ut_hbm, utail_hbm, uj_hbm, usm_hbm),
                (mid_hbm, mt_hbm, mtail_hbm, mj_hbm, msm_hbm)):
            pltpu.sync_copy(ids_hbm, idsv)
            pltpu.sync_copy(tl_hbm, tailv)

            @pl.loop(0, _WLCAP // _LANES)
            def _(i):
                wlc[pl.ds(i * _LANES, _LANES)] = jnp.full(
                    (_LANES,), -1, jnp.int32)

            @pl.loop(0, 8)
            def _(r):
                for q in range(8):
                    wlp[r, pl.ds(q * _LANES, _LANES)] = (
                        _BATCH + ((r * 8 + q) % 8) * _LANES + lanes)

            cnt[0] = 0

            @pl.loop(0, _BATCH // _LANES)
            def _(g):
                idv = idsv[pl.ds(g * _LANES, _LANES)]
                msk = (idv >= lo) & (idv < hi)
                mi = jnp.where(msk, 1, 0).astype(jnp.int32)
                cs = plsc.cumsum(mi)
                base = cnt[0]
                pos16 = base + cs - mi
                plsc.store_scatter(wlc, [pos16], idv, mask=msk)
                plsc.store_scatter(wlp, [pos16 >> 7, pos16 & 127],
                                   g * _LANES + lanes, mask=msk)
                cnt[0] = base + cs[_LANES - 1]

            n = cnt[0]
            ngrp = (n + _LANES - 1) // _LANES

            def c0_of(k):
                return lax.min(lo + k * _W, _CLAMP)

            def fire(k):
                for i in range(4):
                    pltpu.async_copy(
                        embt_hbm.at[i, :, pl.ds(c0_of(k), _W)],
                        slab.at[k & 1, i], sem)

            def drain(k):
                for i in range(4):
                    pltpu.make_async_copy(
                        embt_hbm.at[i, :, pl.ds(c0_of(k), _W)],
                        slab.at[k & 1, i], sem).wait()

            def extract(localv, sel, slotv, gathered):
                for d in range(_DIM):
                    vd = gathered(d, localv, sel)
                    plsc.store_scatter(
                        rowbuf, [slotv * _DIM + d], vd, mask=sel)

            fire(0)

            @pl.loop(0, nch)
            def _(k):
                @pl.when(k + 1 < nch)
                def _():
                    fire(k + 1)
                drain(k)
                c0n = lo + k * _W
                c0 = c0_of(k)
                buf16 = jnp.full((_LANES,), k & 1, jnp.int32)

                @pl.loop(0, ngrp)
                def _(gi):
                    colv = wlc[pl.ds(gi * _LANES, _LANES)]
                    sel = ((colv >= c0n) & (colv < c0n + _W)
                           & (colv < _TAILB))
                    npop = plsc.all_reduce_population_count(sel)[0]

                    @pl.when(npop > 0)
                    def _():
                        slotv = gi * _LANES + lanes
                        localv = jnp.where(sel, colv - c0, 0)
                        extract(localv, sel, slotv,
                                lambda d, lv, s: plsc.load_gather(
                                    slab,
                                    [buf16,
                                     jnp.full((_LANES,), d >> 3, jnp.int32),
                                     jnp.full((_LANES,), d & 7, jnp.int32),
                                     lv],
                                    mask=s))

            @pl.loop(0, ngrp)
            def _(gi):
                colv = wlc[pl.ds(gi * _LANES, _LANES)]
                sel = colv >= _TAILB
                npop = plsc.all_reduce_population_count(sel)[0]

                @pl.when(npop > 0)
                def _():
                    slotv = gi * _LANES + lanes
                    localv = jnp.where(sel, colv - _TAILB, 0)
                    extract(localv, sel, slotv,
                            lambda d, lv, s: plsc.load_gather(
                                tailv, [d * _TAILN + lv], mask=s))

            pltpu.sync_copy(
                rowbuf,
                out_hbm.at[pl.ds(wid * _WLCAP * _DIM, _WLCAP * _DIM)])

            @pl.loop(0, _WLCAP // _LANES)
            def _(i):
                valv[pl.ds(i * _LANES, _LANES)] = (
                    wid * _WLCAP + i * _LANES + lanes)

            sm_copies = []
            for r in range(8):
                sm_copies.append(pltpu.async_copy(
                    valv.at[pl.ds(r * 128, 128)],
                    sm_hbm.at[wlp.at[r]], sem2))
            for c in sm_copies:
                c.wait()

    return k1


def _make_k2():
    mesh, cp = _mesh_and_params(tc_tiling=False)

    @functools.partial(
        pl.kernel,
        mesh=mesh,
        compiler_params=cp,
        out_type=jax.ShapeDtypeStruct((_BATCH,), jnp.float32),
        scratch_types=[
            pltpu.VMEM((4, 128), jnp.int32),
            pltpu.VMEM((4, 128), jnp.int32),
            pltpu.VMEM((_PER_WORKER,), jnp.int32),
            pltpu.VMEM((_PER_WORKER,), jnp.int32),
            pltpu.VMEM((4, 128), jnp.int32),
            pltpu.VMEM((4, 128), jnp.int32),
            pltpu.VMEM((_PER_WORKER, _DIM), jnp.float32),
            pltpu.VMEM((_PER_WORKER, _DIM), jnp.float32),
            pltpu.VMEM((_PER_WORKER,), jnp.float32),
            pltpu.VMEM((_PER_WORKER,), jnp.float32),
            pltpu.VMEM((_PER_WORKER,), jnp.float32),
            pltpu.SemaphoreType.DMA,
            pltpu.SemaphoreType.DMA,
        ],
    )
    def k2(uid_hbm, mid_hbm, uj_hbm, mj_hbm, usm_hbm, msm_hbm,
           ubias_hbm, mbias_hbm, out_hbm, uidx, midx, usv, msv, usx, msx,
           urows, mrows, ubv, mbv, outv, sem, sem2):
        wid = lax.axis_index("s") * _NUM_CORES + lax.axis_index("c")
        base = wid * _PER_WORKER
        lanes = lax.iota(jnp.int32, _LANES)

        pltpu.sync_copy(uid_hbm.at[wid], uidx)
        pltpu.sync_copy(mid_hbm.at[wid], midx)
        pltpu.sync_copy(usm_hbm.at[pl.ds(base, _PER_WORKER)], usv)
        pltpu.sync_copy(msm_hbm.at[pl.ds(base, _PER_WORKER)], msv)

        bias_copies = []
        for j in range(4):
            sl = pl.ds(j * 128, 128)
            bias_copies.append(pltpu.async_copy(
                ubias_hbm.at[uidx.at[j]], ubv.at[sl], sem2))
            bias_copies.append(pltpu.async_copy(
                mbias_hbm.at[midx.at[j]], mbv.at[sl], sem2))

        for j in range(4):
            for h in range(8):
                sl = pl.ds((j * 8 + h) * _LANES, _LANES)
                usx[j, pl.ds(h * _LANES, _LANES)] = usv[sl]
                msx[j, pl.ds(h * _LANES, _LANES)] = msv[sl]

        row_copies = []
        for j in range(4):
            sl = pl.ds(j * 128, 128)
            row_copies.append(pltpu.async_copy(
                uj_hbm.at[usx.at[j]], urows.at[sl], sem))
            row_copies.append(pltpu.async_copy(
                mj_hbm.at[msx.at[j]], mrows.at[sl], sem))
        for rc in row_copies:
            rc.wait()

        @pl.loop(0, _PER_WORKER // _LANES)
        def _(g):
            acc = jnp.zeros((_LANES,), jnp.float32)
            for j in range(_LANES):
                r = g * _LANES + j
                s = (urows[r, 0:_LANES] * mrows[r, 0:_LANES]
                     + urows[r, _LANES:_DIM] * mrows[r, _LANES:_DIM])
                acc = jnp.where(lanes == j, jnp.sum(s), acc)
            outv[pl.ds(g * _LANES, _LANES)] = acc

        for bc in bias_copies:
            bc.wait()
        for h in range(_PER_WORKER // _LANES):
            sl = pl.ds(h * _LANES, _LANES)
            outv[sl] = outv[sl] + ubv[sl] + mbv[sl]

        pltpu.sync_copy(outv, out_hbm.at[pl.ds(base, _PER_WORKER)])

    return k2


def kernel(user_ids, movie_ids, user_emb, movie_emb, user_bias, movie_bias):
    uids = user_ids.astype(jnp.int32)
    mids = movie_ids.astype(jnp.int32)
    ut = user_emb.T
    mt = movie_emb.T
    utail = ut[:, _TAILB:].reshape(-1)
    mtail = mt[:, _TAILB:].reshape(-1)
    ut3 = ut.reshape(4, 8, _V)
    mt3 = mt.reshape(4, 8, _V)
    uj, mj, usm, msm = _make_k1()(uids, mids, ut3, mt3, utail, mtail)
    uj2 = uj.reshape(_NUM_WORKERS * _WLCAP, _DIM)
    mj2 = mj.reshape(_NUM_WORKERS * _WLCAP, _DIM)
    return _make_k2()(
        uids.reshape(_NUM_WORKERS, 4, 128),
        mids.reshape(_NUM_WORKERS, 4, 128),
        uj2, mj2, usm, msm,
        user_bias.reshape(-1), movie_bias.reshape(-1))


# speedup vs baseline: 11.5943x; 4.1734x over previous
import dataclasses
import functools

import jax
import jax.numpy as jnp
from jax import lax
from jax.experimental import pallas as pl
from jax.experimental.pallas import tpu as pltpu
from jax.experimental.pallas import tpu_sc as plsc

_BATCH = 16384
_DIM = 32
_LANES = 16
_NUM_CORES = 2
_NUM_SUBCORES = 16
_NUM_WORKERS = _NUM_CORES * _NUM_SUBCORES
_PER_WORKER = _BATCH // _NUM_WORKERS
_CHUNK = 128
_NUM_CHUNKS = _PER_WORKER // _CHUNK
_ROWS = 250000
_RW = 128


def _make_kernel():
    mesh = plsc.VectorSubcoreMesh(core_axis_name="c", subcore_axis_name="s")
    cp = pltpu.CompilerParams()
    if "needs_layout_passes" in pltpu.CompilerParams.__dataclass_fields__:
        cp = dataclasses.replace(cp, needs_layout_passes=False)

    @functools.partial(
        pl.kernel,
        mesh=mesh,
        compiler_params=cp,
        out_type=jax.ShapeDtypeStruct((_BATCH,), jnp.float32),
        scratch_types=[
            pltpu.VMEM((_NUM_CHUNKS, _CHUNK), jnp.int32),
            pltpu.VMEM((_NUM_CHUNKS, _CHUNK), jnp.int32),
            pltpu.VMEM((_NUM_CHUNKS, _CHUNK), jnp.int32),
            pltpu.VMEM((_NUM_CHUNKS, _CHUNK), jnp.int32),
            pltpu.VMEM((_NUM_CHUNKS, _CHUNK), jnp.int32),
            pltpu.VMEM((_NUM_CHUNKS, _CHUNK), jnp.int32),
            pltpu.VMEM((2, _CHUNK, _RW), jnp.float32),
            pltpu.VMEM((2, _CHUNK, _RW), jnp.float32),
            pltpu.VMEM((_PER_WORKER,), jnp.float32),
            pltpu.VMEM((_PER_WORKER,), jnp.float32),
            pltpu.VMEM((_PER_WORKER,), jnp.float32),
            pltpu.SemaphoreType.DMA,
            pltpu.SemaphoreType.DMA,
        ],
    )
    def k(uid_hbm, mid_hbm, ue_hbm, me_hbm, ubias_hbm, mbias_hbm,
          out_hbm, uidx, midx, urx, mrx, uq, mq, urows, mrows, ubv, mbv,
          outv, sem, sem2):
        wid = lax.axis_index("s") * _NUM_CORES + lax.axis_index("c")
        base = wid * _PER_WORKER
        lanes = lax.iota(jnp.int32, _LANES)

        pltpu.sync_copy(uid_hbm.at[wid], uidx)
        pltpu.sync_copy(mid_hbm.at[wid], midx)

        bias_copies = []
        for j in range(_NUM_CHUNKS):
            sl = pl.ds(j * _CHUNK, _CHUNK)
            bias_copies.append(pltpu.async_copy(
                ubias_hbm.at[uidx.at[j]], ubv.at[sl], sem2))
            bias_copies.append(pltpu.async_copy(
                mbias_hbm.at[midx.at[j]], mbv.at[sl], sem2))

        for j in range(_NUM_CHUNKS):
            for h in range(_CHUNK // _LANES):
                sl = (j, pl.ds(h * _LANES, _LANES))
                iv = uidx[sl]
                uq[sl] = (iv & 3) * _DIM
                urx[sl] = iv >> 2
                iv = midx[sl]
                mq[sl] = (iv & 3) * _DIM
                mrx[sl] = iv >> 2

        def transfers(c, buf):
            return [
                pltpu.make_async_copy(
                    ue_hbm.at[urx.at[c]], urows.at[buf], sem),
                pltpu.make_async_copy(
                    me_hbm.at[mrx.at[c]], mrows.at[buf], sem),
            ]

        def fire(c, buf):
            for t in transfers(c, buf):
                t.start()

        def drain(c, buf):
            for t in transfers(c, buf):
                t.wait()

        def compute(c, buf):
            @pl.loop(0, _CHUNK // _LANES)
            def _(g):
                qu16 = uq[c, pl.ds(g * _LANES, _LANES)]
                qm16 = mq[c, pl.ds(g * _LANES, _LANES)]
                acc = jnp.zeros((_LANES,), jnp.float32)
                for j in range(_LANES):
                    r = g * _LANES + j
                    qa = qu16[j]
                    qb = qm16[j]
                    s = (urows[buf, r, pl.ds(qa, _LANES)]
                         * mrows[buf, r, pl.ds(qb, _LANES)]
                         + urows[buf, r, pl.ds(qa + _LANES, _LANES)]
                         * mrows[buf, r, pl.ds(qb + _LANES, _LANES)])
                    acc = jnp.where(lanes == j, jnp.sum(s), acc)
                osl = pl.ds(c * _CHUNK + g * _LANES, _LANES)
                outv[osl] = acc

        fire(0, 0)
        for c in range(_NUM_CHUNKS):
            if c + 1 < _NUM_CHUNKS:
                fire(c + 1, (c + 1) % 2)
            drain(c, c % 2)
            compute(c, c % 2)

        for bc in bias_copies:
            bc.wait()
        for h in range(_PER_WORKER // _LANES):
            sl = pl.ds(h * _LANES, _LANES)
            outv[sl] = outv[sl] + ubv[sl] + mbv[sl]

        pltpu.sync_copy(outv, out_hbm.at[pl.ds(base, _PER_WORKER)])

    return k


def kernel(user_ids, movie_ids, user_emb, movie_emb, user_bias, movie_bias):
    uids = user_ids.astype(jnp.int32).reshape(_NUM_WORKERS, _NUM_CHUNKS, _CHUNK)
    mids = movie_ids.astype(jnp.int32).reshape(_NUM_WORKERS, _NUM_CHUNKS, _CHUNK)
    ue = user_emb.reshape(_ROWS, _RW)
    me = movie_emb.reshape(_ROWS, _RW)
    ubias = user_bias.reshape(-1)
    mbias = movie_bias.reshape(-1)
    k = _make_kernel()
    return k(uids, mids, ue, me, ubias, mbias)
